# baseline (speedup 1.0000x reference)
import functools

import jax
import jax.numpy as jnp
from jax import lax
from jax.experimental import pallas as pl
from jax.experimental.pallas import tpu as pltpu
from jax.experimental.pallas import tpu_sc as plsc

E = 320000
D = 128
V = 20

NW = 32
PER_W = E // NW
GROUP = 80
NBUF = 5
NOUTER = PER_W // (GROUP * NBUF)

CB = 3200
NCB = E // CB


def _combos_body(e0_ref, e1_ref, e2_ref, out_ref):
    i0 = pl.program_id(0)
    t01 = e0_ref[i0, :][None, :] + e1_ref[...]
    blk = t01[:, None, :] + e2_ref[...][None, :, :]
    out_ref[...] = blk.reshape(V * V, D)


def _combos(emb0, emb1, emb2):
    return pl.pallas_call(
        _combos_body,
        grid=(V,),
        in_specs=[
            pl.BlockSpec((V, D), lambda i: (0, 0)),
            pl.BlockSpec((V, D), lambda i: (0, 0)),
            pl.BlockSpec((V, D), lambda i: (0, 0)),
        ],
        out_specs=pl.BlockSpec((V * V, D), lambda i: (i, 0)),
        out_shape=jax.ShapeDtypeStruct((V * V * V, D), jnp.float32),
    )(emb0, emb1, emb2)


def _cidx_body(attr_ref, out_ref):
    a = attr_ref[...]
    col = lax.broadcasted_iota(jnp.int32, (1, 3), 1)
    w = jnp.where(col == 0, 400, jnp.where(col == 1, 20, 1))
    out_ref[...] = jnp.sum(a * w, axis=1)[None, None, :]


def _cidx(edge_attr):
    fused = pl.pallas_call(
        _cidx_body,
        grid=(NCB,),
        in_specs=[pl.BlockSpec((CB, 3), lambda i: (i, 0))],
        out_specs=pl.BlockSpec((1, 1, CB), lambda i: (i, 0, 0)),
        out_shape=jax.ShapeDtypeStruct((NCB, 1, CB), jnp.int32),
    )(edge_attr)
    return fused.reshape(E)


@functools.partial(
    pl.kernel,
    mesh=plsc.VectorSubcoreMesh(core_axis_name="c", subcore_axis_name="s"),
    out_type=jax.ShapeDtypeStruct((E, D), jnp.float32),
    scratch_types=(
        [pltpu.VMEM((PER_W,), jnp.int32)]
        + [pltpu.VMEM((GROUP, D), jnp.float32)] * NBUF
        + [pltpu.SemaphoreType.DMA] * (1 + 2 * NBUF)
    ),
)
def _sc_gather(cidx_hbm, combos_hbm, out_hbm, cidx_v, *bufs_and_sems):
    rows = bufs_and_sems[:NBUF]
    isem = bufs_and_sems[NBUF]
    gsem = bufs_and_sems[NBUF + 1:2 * NBUF + 1]
    ssem = bufs_and_sems[2 * NBUF + 1:]
    wid = lax.axis_index("s") * 2 + lax.axis_index("c")
    base = wid * PER_W

    pltpu.async_copy(cidx_hbm.at[pl.ds(base, PER_W)], cidx_v, isem).wait()

    def outer_body(o, carry):
        goff = pl.multiple_of(o * (GROUP * NBUF), GROUP * NBUF)
        gcps = []
        for b in range(NBUF):
            @pl.when(o > 0)
            def _(b=b):
                pltpu.make_async_copy(
                    rows[b], out_hbm.at[pl.ds(0, GROUP)], ssem[b]).wait()
            cidx_sl = cidx_v.at[pl.ds(goff + b * GROUP, GROUP)]
            gcps.append(pltpu.async_copy(
                combos_hbm.at[cidx_sl], rows[b], gsem[b]))
        for b in range(NBUF):
            gcps[b].wait()
            pltpu.async_copy(
                rows[b], out_hbm.at[pl.ds(base + goff + b * GROUP, GROUP)],
                ssem[b])
        return carry

    lax.fori_loop(0, NOUTER, outer_body, 0)
    for b in range(NBUF):
        pltpu.make_async_copy(
            rows[b], out_hbm.at[pl.ds(0, GROUP)], ssem[b]).wait()


def kernel(edge_attr, emb0, emb1, emb2):
    combos = _combos(emb0, emb1, emb2)
    cidx = _cidx(edge_attr)
    return _sc_gather(cidx, combos)


# speedup vs baseline: 2.5166x; 2.5166x over previous
import functools

import jax
import jax.numpy as jnp
from jax import lax
from jax.experimental import pallas as pl
from jax.experimental.pallas import tpu as pltpu
from jax.experimental.pallas import tpu_sc as plsc

E = 320000
D = 128
V = 20

NW = 32
PER_W = E // NW
GROUP = 80
NBUF = 5
NOUTER = PER_W // (GROUP * NBUF)

CB = 3200
NCB = E // CB


def _combos_body(e0_ref, e1_ref, e2_ref, out_ref):
    i0 = pl.program_id(0)
    t01 = e0_ref[i0, :][None, :] + e1_ref[...]
    blk = t01[:, None, :] + e2_ref[...][None, :, :]
    out_ref[...] = blk.reshape(V * V, D)


def _combos(emb0, emb1, emb2):
    return pl.pallas_call(
        _combos_body,
        grid=(V,),
        in_specs=[
            pl.BlockSpec((V, D), lambda i: (0, 0)),
            pl.BlockSpec((V, D), lambda i: (0, 0)),
            pl.BlockSpec((V, D), lambda i: (0, 0)),
        ],
        out_specs=pl.BlockSpec((V * V, D), lambda i: (i, 0)),
        out_shape=jax.ShapeDtypeStruct((V * V * V, D), jnp.float32),
    )(emb0, emb1, emb2)


@functools.partial(
    pl.kernel,
    mesh=plsc.VectorSubcoreMesh(core_axis_name="c", subcore_axis_name="s"),
    out_type=jax.ShapeDtypeStruct((E, D), jnp.float32),
    scratch_types=(
        [pltpu.VMEM((PER_W,), jnp.int32)] * 4
        + [pltpu.VMEM((GROUP, D), jnp.float32)] * NBUF
        + [pltpu.SemaphoreType.DMA] * (1 + 2 * NBUF)
    ),
)
def _sc_gather(attr0_hbm, attr1_hbm, attr2_hbm, combos_hbm, out_hbm,
               a0_v, a1_v, a2_v, cidx_v, *bufs_and_sems):
    rows = bufs_and_sems[:NBUF]
    isem = bufs_and_sems[NBUF]
    gsem = bufs_and_sems[NBUF + 1:2 * NBUF + 1]
    ssem = bufs_and_sems[2 * NBUF + 1:]
    wid = lax.axis_index("s") * 2 + lax.axis_index("c")
    base = wid * PER_W

    cps = [pltpu.async_copy(a.at[pl.ds(base, PER_W)], v, isem)
           for a, v in ((attr0_hbm, a0_v), (attr1_hbm, a1_v),
                        (attr2_hbm, a2_v))]
    for cp in cps:
        cp.wait()

    def fuse_body(j, carry):
        sl = pl.ds(pl.multiple_of(j * 16, 16), 16)
        cidx_v[sl] = a0_v[sl] * 400 + a1_v[sl] * 20 + a2_v[sl]
        return carry

    lax.fori_loop(0, PER_W // 16, fuse_body, 0)

    def outer_body(o, carry):
        goff = pl.multiple_of(o * (GROUP * NBUF), GROUP * NBUF)
        gcps = []
        for b in range(NBUF):
            @pl.when(o > 0)
            def _(b=b):
                pltpu.make_async_copy(
                    rows[b], out_hbm.at[pl.ds(0, GROUP)], ssem[b]).wait()
            cidx_sl = cidx_v.at[pl.ds(goff + b * GROUP, GROUP)]
            gcps.append(pltpu.async_copy(
                combos_hbm.at[cidx_sl], rows[b], gsem[b]))
        for b in range(NBUF):
            gcps[b].wait()
            pltpu.async_copy(
                rows[b], out_hbm.at[pl.ds(base + goff + b * GROUP, GROUP)],
                ssem[b])
        return carry

    lax.fori_loop(0, NOUTER, outer_body, 0)
    for b in range(NBUF):
        pltpu.make_async_copy(
            rows[b], out_hbm.at[pl.ds(0, GROUP)], ssem[b]).wait()


def kernel(edge_attr, emb0, emb1, emb2):
    combos = _combos(emb0, emb1, emb2)
    attr_t = edge_attr.T
    return _sc_gather(attr_t[0], attr_t[1], attr_t[2], combos)
